# baseline (speedup 1.0000x reference)
import jax
import jax.numpy as jnp
from jax import lax
from jax.experimental import pallas as pl
from jax.experimental.pallas import tpu as pltpu
from jax.experimental.pallas import tpu_sc as plsc

NN = 10000
EE = 320000
DD = 128
RR = 4
GG = 8
LL = 3

NC = 2
NS = 16
NW = NC * NS

CH = 128
NCHUNK = EE // CH
CPW = -(-NCHUNK // NW)
NP = 10240
RPT = NP // NS

BN = 1000
NB = NN // BN


NBR = 2
NBI = 4


def _edge_agg_body(t_hbm, idx_hbm, out_hbm,
                   ibufs, rowbufs, acc, semi, semg, sems):
    c = lax.axis_index("c")
    s = lax.axis_index("s")
    w = s * NC + c

    def idx_start(k, bi):
        pltpu.async_copy(idx_hbm.at[w + NW * k], ibufs[bi], semi[bi])

    def idx_wait(k, bi):
        pltpu.make_async_copy(idx_hbm.at[w + NW * k], ibufs[bi],
                              semi[bi]).wait()

    def gather_start(bi, b):
        pltpu.async_copy(t_hbm.at[ibufs[bi].at[0]], rowbufs[b], semg[b])

    def gather_wait(bi, b):
        pltpu.make_async_copy(t_hbm.at[ibufs[bi].at[0]], rowbufs[b],
                              semg[b]).wait()

    def scatter_start(bi, b):
        pltpu.async_copy(rowbufs[b], acc.at[ibufs[bi].at[1]], sems[b],
                         add=True)

    def scatter_wait(bi, b):
        pltpu.make_async_copy(rowbufs[b], acc.at[ibufs[bi].at[1]],
                              sems[b]).wait()

    def valid(k):
        return w + NW * k < NCHUNK

    idx_start(0, 0)
    idx_start(1, 1)
    idx_start(2, 2)
    zero16 = jnp.zeros((16,), jnp.float32)

    def zrow(i, _):
        for j in range(DD // 16):
            rowbufs[1][i, pl.ds(j * 16, 16)] = zero16
        return 0

    lax.fori_loop(0, CH, zrow, 0)
    idx_wait(0, 0)
    gather_start(0, 0)
    for k in range(RPT // CH):
        pltpu.sync_copy(rowbufs[1], acc.at[pl.ds(s * RPT + k * CH, CH)])
    plsc.subcore_barrier()

    def body(j, _):
        for u in range(NBI):
            k = NBI * j + u
            b, bn = u % NBR, (u + 1) % NBR
            bi, bim, bin_, bip = u, (u - 1) % NBI, (u + 1) % NBI, (u + 3) % NBI

            @pl.when(valid(k))
            def _():
                gather_wait(bi, b)
                scatter_start(bi, b)

            @pl.when(jnp.logical_and(k >= 1, valid(k - 1)))
            def _():
                scatter_wait(bim, bn)

            @pl.when(valid(k + 1))
            def _():
                idx_wait(k + 1, bin_)
                gather_start(bin_, bn)

            @pl.when(valid(k + 3))
            def _():
                idx_start(k + 3, bip)

        return 0

    JM = -(-CPW // NBI)
    lax.fori_loop(0, JM, body, 0)

    kL = NBI * JM - 1

    @pl.when(valid(kL))
    def _():
        scatter_wait(kL % NBI, kL % NBR)

    plsc.subcore_barrier()
    pltpu.sync_copy(acc.at[pl.ds(s * RPT, RPT)],
                    out_hbm.at[c, pl.ds(s * RPT, RPT)])


_edge_agg = pl.kernel(
    _edge_agg_body,
    out_type=jax.ShapeDtypeStruct((NC, NP, DD), jnp.float32),
    mesh=plsc.VectorSubcoreMesh(core_axis_name="c", subcore_axis_name="s",
                                num_cores=NC, num_subcores=NS),
    scratch_types=[
        [pltpu.VMEM((2, CH), jnp.int32)] * NBI,
        [pltpu.VMEM((CH, DD), jnp.float32)] * NBR,
        pltpu.VMEM_SHARED((NP, DD), jnp.float32),
        [pltpu.SemaphoreType.DMA] * NBI,
        [pltpu.SemaphoreType.DMA] * NBR,
        [pltpu.SemaphoreType.DMA] * NBR,
    ],
)


def _xform(h, wr_ref, ws_ref, b_ref, t_ref, s_ref):
    for r in range(RR):
        t_ref[r] = jnp.dot(h, wr_ref[r * DD:(r + 1) * DD, :],
                           preferred_element_type=jnp.float32)
    s_ref[...] = jnp.dot(h, ws_ref[...],
                         preferred_element_type=jnp.float32) + b_ref[...]


def _l0_body(x_ref, wr_ref, ws_ref, b_ref, t_ref, s_ref):
    _xform(x_ref[...], wr_ref, ws_ref, b_ref, t_ref, s_ref)


def _li_body(p_ref, sp_ref, wr_ref, ws_ref, b_ref, t_ref, s_ref):
    h = jnp.maximum(p_ref[0] + p_ref[1] + sp_ref[...], 0.0)
    _xform(h, wr_ref, ws_ref, b_ref, t_ref, s_ref)


def _readout_body(n2g_ref, p_ref, sp_ref, out_ref):
    h = jnp.maximum(p_ref[0] + p_ref[1] + sp_ref[...], 0.0)
    n2g = n2g_ref[0, 0, :]
    onehot = (n2g[:, None] == lax.broadcasted_iota(jnp.int32, (BN, GG), 1)
              ).astype(jnp.float32)
    contrib = lax.dot_general(onehot, h, (((0,), (0,)), ((), ())),
                              preferred_element_type=jnp.float32)

    @pl.when(pl.program_id(0) == 0)
    def _():
        out_ref[...] = jnp.zeros_like(out_ref)

    out_ref[...] += contrib


_W_SPECS = [
    pl.BlockSpec((RR * DD, DD), lambda i: (0, 0)),
    pl.BlockSpec((DD, DD), lambda i: (0, 0)),
    pl.BlockSpec((1, DD), lambda i: (0, 0)),
]
_TS_OUT = dict(
    out_specs=[
        pl.BlockSpec((RR, BN, DD), lambda i: (0, i, 0)),
        pl.BlockSpec((BN, DD), lambda i: (i, 0)),
    ],
    out_shape=[
        jax.ShapeDtypeStruct((RR, NN, DD), jnp.float32),
        jax.ShapeDtypeStruct((NN, DD), jnp.float32),
    ],
)

_l0 = pl.pallas_call(
    _l0_body,
    grid=(NB,),
    in_specs=[pl.BlockSpec((BN, DD), lambda i: (i, 0))] + _W_SPECS,
    **_TS_OUT,
)

_li = pl.pallas_call(
    _li_body,
    grid=(NB,),
    in_specs=[
        pl.BlockSpec((NC, BN, DD), lambda i: (0, i, 0)),
        pl.BlockSpec((BN, DD), lambda i: (i, 0)),
    ] + _W_SPECS,
    **_TS_OUT,
)

_readout = pl.pallas_call(
    _readout_body,
    grid=(NB,),
    in_specs=[
        pl.BlockSpec((1, 1, BN), lambda i: (i, 0, 0)),
        pl.BlockSpec((NC, BN, DD), lambda i: (0, i, 0)),
        pl.BlockSpec((BN, DD), lambda i: (i, 0)),
    ],
    out_specs=pl.BlockSpec((GG, DD), lambda i: (0, 0)),
    out_shape=jax.ShapeDtypeStruct((GG, DD), jnp.float32),
)


def kernel(x, edge_index, edge_type, node2graph,
           Wrel0, brel0, Wself0, bself0,
           Wrel1, brel1, Wself1, bself1,
           Wrel2, brel2, Wself2, bself2):
    src = edge_index[0].astype(jnp.int32)
    dst = edge_index[1].astype(jnp.int32)
    gidx = (edge_type.astype(jnp.int32) * NN + src).reshape(NCHUNK, CH)
    idx2 = jnp.stack([gidx, dst.reshape(NCHUNK, CH)], axis=1)
    n2g3d = node2graph.astype(jnp.int32).reshape(NB, 1, BN)

    Wrels = (Wrel0, Wrel1, Wrel2)
    Wselfs = (Wself0, Wself1, Wself2)
    biases = tuple((br + bs).reshape(1, DD)
                   for br, bs in ((brel0, bself0), (brel1, bself1),
                                  (brel2, bself2)))

    T, S = _l0(x, Wrels[0], Wselfs[0], biases[0])
    for i in range(1, LL):
        P = _edge_agg(T.reshape(RR * NN, DD), idx2)
        T, S = _li(P, S, Wrels[i], Wselfs[i], biases[i])
    P = _edge_agg(T.reshape(RR * NN, DD), idx2)
    return _readout(n2g3d, P, S)


# speedup vs baseline: 1.0309x; 1.0309x over previous
import jax
import jax.numpy as jnp
from jax import lax
from jax.experimental import pallas as pl
from jax.experimental.pallas import tpu as pltpu
from jax.experimental.pallas import tpu_sc as plsc

NN = 10000
EE = 320000
DD = 128
RR = 4
GG = 8
LL = 3

NC = 2
NS = 16
NW = NC * NS

CH = 128
NCHUNK = EE // CH
CPW = -(-NCHUNK // NW)
NP = 10240
RPT = NP // NS

BN = 2000
NB = NN // BN


NBR = 2
NBI = 4


def _edge_agg_body(t_hbm, idx_hbm, out_hbm,
                   ibufs, rowbufs, acc, semi, semg, sems):
    c = lax.axis_index("c")
    s = lax.axis_index("s")
    w = s * NC + c

    def idx_start(k, bi):
        pltpu.async_copy(idx_hbm.at[w + NW * k], ibufs[bi], semi[bi])

    def idx_wait(k, bi):
        pltpu.make_async_copy(idx_hbm.at[w + NW * k], ibufs[bi],
                              semi[bi]).wait()

    def gather_start(bi, b):
        pltpu.async_copy(t_hbm.at[ibufs[bi].at[0]], rowbufs[b], semg[b])

    def gather_wait(bi, b):
        pltpu.make_async_copy(t_hbm.at[ibufs[bi].at[0]], rowbufs[b],
                              semg[b]).wait()

    def scatter_start(bi, b):
        pltpu.async_copy(rowbufs[b], acc.at[ibufs[bi].at[1]], sems[b],
                         add=True)

    def scatter_wait(bi, b):
        pltpu.make_async_copy(rowbufs[b], acc.at[ibufs[bi].at[1]],
                              sems[b]).wait()

    def valid(k):
        return w + NW * k < NCHUNK

    idx_start(0, 0)
    idx_start(1, 1)
    idx_start(2, 2)
    zero16 = jnp.zeros((16,), jnp.float32)

    def zrow(i, _):
        for j in range(DD // 16):
            rowbufs[1][i, pl.ds(j * 16, 16)] = zero16
        return 0

    lax.fori_loop(0, CH, zrow, 0)
    idx_wait(0, 0)
    gather_start(0, 0)
    for k in range(RPT // CH):
        pltpu.sync_copy(rowbufs[1], acc.at[pl.ds(s * RPT + k * CH, CH)])
    plsc.subcore_barrier()

    def body(j, _):
        for u in range(NBI):
            k = NBI * j + u
            b, bn = u % NBR, (u + 1) % NBR
            bi, bim, bin_, bip = u, (u - 1) % NBI, (u + 1) % NBI, (u + 3) % NBI

            @pl.when(valid(k))
            def _():
                gather_wait(bi, b)
                scatter_start(bi, b)

            @pl.when(jnp.logical_and(k >= 1, valid(k - 1)))
            def _():
                scatter_wait(bim, bn)

            @pl.when(valid(k + 1))
            def _():
                idx_wait(k + 1, bin_)
                gather_start(bin_, bn)

            @pl.when(valid(k + 3))
            def _():
                idx_start(k + 3, bip)

        return 0

    JM = -(-CPW // NBI)
    lax.fori_loop(0, JM, body, 0)

    kL = NBI * JM - 1

    @pl.when(valid(kL))
    def _():
        scatter_wait(kL % NBI, kL % NBR)

    plsc.subcore_barrier()
    pltpu.sync_copy(acc.at[pl.ds(s * RPT, RPT)],
                    out_hbm.at[c, pl.ds(s * RPT, RPT)])


_edge_agg = pl.kernel(
    _edge_agg_body,
    out_type=jax.ShapeDtypeStruct((NC, NP, DD), jnp.float32),
    mesh=plsc.VectorSubcoreMesh(core_axis_name="c", subcore_axis_name="s",
                                num_cores=NC, num_subcores=NS),
    scratch_types=[
        [pltpu.VMEM((2, CH), jnp.int32)] * NBI,
        [pltpu.VMEM((CH, DD), jnp.float32)] * NBR,
        pltpu.VMEM_SHARED((NP, DD), jnp.float32),
        [pltpu.SemaphoreType.DMA] * NBI,
        [pltpu.SemaphoreType.DMA] * NBR,
        [pltpu.SemaphoreType.DMA] * NBR,
    ],
)


def _tmuls(h, wr_ref, t_ref):
    for r in range(RR):
        t_ref[r] = jnp.dot(h, wr_ref[r * DD:(r + 1) * DD, :],
                           preferred_element_type=jnp.float32)


def _l0t_body(x_ref, wr_ref, t_ref):
    _tmuls(x_ref[...], wr_ref, t_ref)


def _lit_body(p_ref, sp_ref, wr_ref, t_ref):
    h = jnp.maximum(p_ref[0] + p_ref[1] + sp_ref[...], 0.0)
    _tmuls(h, wr_ref, t_ref)


def _s0_body(x_ref, ws_ref, b_ref, s_ref):
    s_ref[...] = jnp.dot(x_ref[...], ws_ref[...],
                         preferred_element_type=jnp.float32) + b_ref[...]


def _si_body(p_ref, sp_ref, ws_ref, b_ref, s_ref):
    h = jnp.maximum(p_ref[0] + p_ref[1] + sp_ref[...], 0.0)
    s_ref[...] = jnp.dot(h, ws_ref[...],
                         preferred_element_type=jnp.float32) + b_ref[...]


def _readout_body(n2g_ref, p_ref, sp_ref, out_ref):
    h = jnp.maximum(p_ref[0] + p_ref[1] + sp_ref[...], 0.0)
    n2g = n2g_ref[0, 0, :]
    onehot = (n2g[:, None] == lax.broadcasted_iota(jnp.int32, (BN, GG), 1)
              ).astype(jnp.float32)
    contrib = lax.dot_general(onehot, h, (((0,), (0,)), ((), ())),
                              preferred_element_type=jnp.float32)

    @pl.when(pl.program_id(0) == 0)
    def _():
        out_ref[...] = jnp.zeros_like(out_ref)

    out_ref[...] += contrib


_WREL_SPEC = pl.BlockSpec((RR * DD, DD), lambda i: (0, 0))
_WS_SPECS = [
    pl.BlockSpec((DD, DD), lambda i: (0, 0)),
    pl.BlockSpec((1, DD), lambda i: (0, 0)),
]
_H_SPEC = pl.BlockSpec((BN, DD), lambda i: (i, 0))
_P_SPEC = pl.BlockSpec((NC, BN, DD), lambda i: (0, i, 0))
_T_OUT = dict(
    out_specs=pl.BlockSpec((RR, BN, DD), lambda i: (0, i, 0)),
    out_shape=jax.ShapeDtypeStruct((RR, NN, DD), jnp.float32),
)
_S_OUT = dict(
    out_specs=_H_SPEC,
    out_shape=jax.ShapeDtypeStruct((NN, DD), jnp.float32),
)

_l0t = pl.pallas_call(_l0t_body, grid=(NB,),
                      in_specs=[_H_SPEC, _WREL_SPEC], **_T_OUT)

_lit = pl.pallas_call(_lit_body, grid=(NB,),
                      in_specs=[_P_SPEC, _H_SPEC, _WREL_SPEC], **_T_OUT)

_s0 = pl.pallas_call(_s0_body, grid=(NB,),
                     in_specs=[_H_SPEC] + _WS_SPECS, **_S_OUT)

_si = pl.pallas_call(_si_body, grid=(NB,),
                     in_specs=[_P_SPEC, _H_SPEC] + _WS_SPECS, **_S_OUT)

_readout = pl.pallas_call(
    _readout_body,
    grid=(NB,),
    in_specs=[
        pl.BlockSpec((1, 1, BN), lambda i: (i, 0, 0)),
        pl.BlockSpec((NC, BN, DD), lambda i: (0, i, 0)),
        pl.BlockSpec((BN, DD), lambda i: (i, 0)),
    ],
    out_specs=pl.BlockSpec((GG, DD), lambda i: (0, 0)),
    out_shape=jax.ShapeDtypeStruct((GG, DD), jnp.float32),
)


def kernel(x, edge_index, edge_type, node2graph,
           Wrel0, brel0, Wself0, bself0,
           Wrel1, brel1, Wself1, bself1,
           Wrel2, brel2, Wself2, bself2):
    src = edge_index[0].astype(jnp.int32)
    dst = edge_index[1].astype(jnp.int32)
    gidx = (edge_type.astype(jnp.int32) * NN + src).reshape(NCHUNK, CH)
    idx2 = jnp.stack([gidx, dst.reshape(NCHUNK, CH)], axis=1)
    n2g3d = node2graph.astype(jnp.int32).reshape(NB, 1, BN)

    Wrels = (Wrel0, Wrel1, Wrel2)
    Wselfs = (Wself0, Wself1, Wself2)
    biases = tuple((br + bs).reshape(1, DD)
                   for br, bs in ((brel0, bself0), (brel1, bself1),
                                  (brel2, bself2)))

    T = _l0t(x, Wrels[0])
    S = _s0(x, Wselfs[0], biases[0])
    for i in range(1, LL):
        P = _edge_agg(T.reshape(RR * NN, DD), idx2)
        T = _lit(P, S, Wrels[i])
        S = _si(P, S, Wselfs[i], biases[i])
    P = _edge_agg(T.reshape(RR * NN, DD), idx2)
    return _readout(n2g3d, P, S)


# speedup vs baseline: 1.0724x; 1.0403x over previous
import jax
import jax.numpy as jnp
from jax import lax
from jax.experimental import pallas as pl
from jax.experimental.pallas import tpu as pltpu
from jax.experimental.pallas import tpu_sc as plsc

NN = 10000
EE = 320000
DD = 128
RR = 4
GG = 8
LL = 3

NC = 2
NS = 16
NW = NC * NS

CH = 128
NCHUNK = EE // CH
CPW = -(-NCHUNK // NW)
NP = 10240
RPT = NP // NS

BN = 2000
NB = NN // BN


NBR = 2
NBI = 4


def _edge_agg_body(t_hbm, ei_hbm, et_hbm, out_hbm,
                   ibufs, sbufs, rowbufs, acc, semi, semg, sems):
    c = lax.axis_index("c")
    s = lax.axis_index("s")
    w = s * NC + c

    def idx_start(k, bi):
        base = (w + NW * k) * CH
        pltpu.async_copy(et_hbm.at[pl.ds(base, CH)], ibufs[bi].at[0],
                         semi[bi])
        pltpu.async_copy(ei_hbm.at[0, pl.ds(base, CH)], sbufs[bi], semi[bi])
        pltpu.async_copy(ei_hbm.at[1, pl.ds(base, CH)], ibufs[bi].at[1],
                         semi[bi])

    def idx_wait(k, bi):
        base = (w + NW * k) * CH
        pltpu.make_async_copy(et_hbm.at[pl.ds(base, CH)], ibufs[bi].at[0],
                              semi[bi]).wait()
        pltpu.make_async_copy(ei_hbm.at[0, pl.ds(base, CH)], sbufs[bi],
                              semi[bi]).wait()
        pltpu.make_async_copy(ei_hbm.at[1, pl.ds(base, CH)], ibufs[bi].at[1],
                              semi[bi]).wait()
        for j in range(CH // 16):
            sl = pl.ds(j * 16, 16)
            ibufs[bi][0, sl] = ibufs[bi][0, sl] * NN + sbufs[bi][sl]

    def gather_start(bi, b):
        pltpu.async_copy(t_hbm.at[ibufs[bi].at[0]], rowbufs[b], semg[b])

    def gather_wait(bi, b):
        pltpu.make_async_copy(t_hbm.at[ibufs[bi].at[0]], rowbufs[b],
                              semg[b]).wait()

    def scatter_start(bi, b):
        pltpu.async_copy(rowbufs[b], acc.at[ibufs[bi].at[1]], sems[b],
                         add=True)

    def scatter_wait(bi, b):
        pltpu.make_async_copy(rowbufs[b], acc.at[ibufs[bi].at[1]],
                              sems[b]).wait()

    def valid(k):
        return w + NW * k < NCHUNK

    idx_start(0, 0)
    idx_start(1, 1)
    idx_start(2, 2)
    zero16 = jnp.zeros((16,), jnp.float32)

    def zrow(i, _):
        for j in range(DD // 16):
            rowbufs[1][i, pl.ds(j * 16, 16)] = zero16
        return 0

    lax.fori_loop(0, CH, zrow, 0)
    idx_wait(0, 0)
    gather_start(0, 0)
    for k in range(RPT // CH):
        pltpu.sync_copy(rowbufs[1], acc.at[pl.ds(s * RPT + k * CH, CH)])
    plsc.subcore_barrier()

    def body(j, _):
        for u in range(NBI):
            k = NBI * j + u
            b, bn = u % NBR, (u + 1) % NBR
            bi, bim, bin_, bip = u, (u - 1) % NBI, (u + 1) % NBI, (u + 3) % NBI

            @pl.when(valid(k))
            def _():
                gather_wait(bi, b)
                scatter_start(bi, b)

            @pl.when(jnp.logical_and(k >= 1, valid(k - 1)))
            def _():
                scatter_wait(bim, bn)

            @pl.when(valid(k + 1))
            def _():
                idx_wait(k + 1, bin_)
                gather_start(bin_, bn)

            @pl.when(valid(k + 3))
            def _():
                idx_start(k + 3, bip)

        return 0

    JM = -(-CPW // NBI)
    lax.fori_loop(0, JM, body, 0)

    kL = NBI * JM - 1

    @pl.when(valid(kL))
    def _():
        scatter_wait(kL % NBI, kL % NBR)

    plsc.subcore_barrier()
    pltpu.sync_copy(acc.at[pl.ds(s * RPT, RPT)],
                    out_hbm.at[c, pl.ds(s * RPT, RPT)])


_edge_agg = pl.kernel(
    _edge_agg_body,
    out_type=jax.ShapeDtypeStruct((NC, NP, DD), jnp.float32),
    mesh=plsc.VectorSubcoreMesh(core_axis_name="c", subcore_axis_name="s",
                                num_cores=NC, num_subcores=NS),
    scratch_types=[
        [pltpu.VMEM((2, CH), jnp.int32)] * NBI,
        [pltpu.VMEM((CH,), jnp.int32)] * NBI,
        [pltpu.VMEM((CH, DD), jnp.float32)] * NBR,
        pltpu.VMEM_SHARED((NP, DD), jnp.float32),
        [pltpu.SemaphoreType.DMA] * NBI,
        [pltpu.SemaphoreType.DMA] * NBR,
        [pltpu.SemaphoreType.DMA] * NBR,
    ],
)


def _tmuls(h, wr_ref, t_ref):
    for r in range(RR):
        t_ref[r] = jnp.dot(h, wr_ref[r * DD:(r + 1) * DD, :],
                           preferred_element_type=jnp.float32)


def _l0t_body(x_ref, wr_ref, t_ref):
    _tmuls(x_ref[...], wr_ref, t_ref)


def _lit_body(p_ref, sp_ref, wr_ref, t_ref):
    h = jnp.maximum(p_ref[0] + p_ref[1] + sp_ref[...], 0.0)
    _tmuls(h, wr_ref, t_ref)


def _s0_body(x_ref, ws_ref, b_ref, s_ref):
    s_ref[...] = jnp.dot(x_ref[...], ws_ref[...],
                         preferred_element_type=jnp.float32) + b_ref[...]


def _si_body(p_ref, sp_ref, ws_ref, b_ref, s_ref):
    h = jnp.maximum(p_ref[0] + p_ref[1] + sp_ref[...], 0.0)
    s_ref[...] = jnp.dot(h, ws_ref[...],
                         preferred_element_type=jnp.float32) + b_ref[...]


def _readout_body(n2g_ref, p_ref, sp_ref, out_ref):
    h = jnp.maximum(p_ref[0] + p_ref[1] + sp_ref[...], 0.0)
    n2g = n2g_ref[0, 0, :]
    onehot = (n2g[:, None] == lax.broadcasted_iota(jnp.int32, (BN, GG), 1)
              ).astype(jnp.float32)
    contrib = lax.dot_general(onehot, h, (((0,), (0,)), ((), ())),
                              preferred_element_type=jnp.float32)

    @pl.when(pl.program_id(0) == 0)
    def _():
        out_ref[...] = jnp.zeros_like(out_ref)

    out_ref[...] += contrib


_WREL_SPEC = pl.BlockSpec((RR * DD, DD), lambda i: (0, 0))
_WS_SPECS = [
    pl.BlockSpec((DD, DD), lambda i: (0, 0)),
    pl.BlockSpec((1, DD), lambda i: (0, 0)),
]
_H_SPEC = pl.BlockSpec((BN, DD), lambda i: (i, 0))
_P_SPEC = pl.BlockSpec((NC, BN, DD), lambda i: (0, i, 0))
_T_OUT = dict(
    out_specs=pl.BlockSpec((RR, BN, DD), lambda i: (0, i, 0)),
    out_shape=jax.ShapeDtypeStruct((RR, NN, DD), jnp.float32),
)
_S_OUT = dict(
    out_specs=_H_SPEC,
    out_shape=jax.ShapeDtypeStruct((NN, DD), jnp.float32),
)

_l0t = pl.pallas_call(_l0t_body, grid=(NB,),
                      in_specs=[_H_SPEC, _WREL_SPEC], **_T_OUT)

_lit = pl.pallas_call(_lit_body, grid=(NB,),
                      in_specs=[_P_SPEC, _H_SPEC, _WREL_SPEC], **_T_OUT)

_s0 = pl.pallas_call(_s0_body, grid=(NB,),
                     in_specs=[_H_SPEC] + _WS_SPECS, **_S_OUT)

_si = pl.pallas_call(_si_body, grid=(NB,),
                     in_specs=[_P_SPEC, _H_SPEC] + _WS_SPECS, **_S_OUT)

_readout = pl.pallas_call(
    _readout_body,
    grid=(NB,),
    in_specs=[
        pl.BlockSpec((1, 1, BN), lambda i: (i, 0, 0)),
        pl.BlockSpec((NC, BN, DD), lambda i: (0, i, 0)),
        pl.BlockSpec((BN, DD), lambda i: (i, 0)),
    ],
    out_specs=pl.BlockSpec((GG, DD), lambda i: (0, 0)),
    out_shape=jax.ShapeDtypeStruct((GG, DD), jnp.float32),
)


def kernel(x, edge_index, edge_type, node2graph,
           Wrel0, brel0, Wself0, bself0,
           Wrel1, brel1, Wself1, bself1,
           Wrel2, brel2, Wself2, bself2):
    ei = edge_index.astype(jnp.int32)
    et = edge_type.astype(jnp.int32)
    n2g3d = node2graph.astype(jnp.int32).reshape(NB, 1, BN)

    Wrels = (Wrel0, Wrel1, Wrel2)
    Wselfs = (Wself0, Wself1, Wself2)
    biases = tuple((br + bs).reshape(1, DD)
                   for br, bs in ((brel0, bself0), (brel1, bself1),
                                  (brel2, bself2)))

    T = _l0t(x, Wrels[0])
    S = _s0(x, Wselfs[0], biases[0])
    for i in range(1, LL):
        P = _edge_agg(T.reshape(RR * NN, DD), ei, et)
        T = _lit(P, S, Wrels[i])
        S = _si(P, S, Wselfs[i], biases[i])
    P = _edge_agg(T.reshape(RR * NN, DD), ei, et)
    return _readout(n2g3d, P, S)
